# baseline (speedup 1.0000x reference)
import jax
import jax.numpy as jnp
from jax.experimental import pallas as pl
from jax.experimental.pallas import tpu as pltpu


def _pos_embed_kernel(x_ref, col_ref, row_ref, out_ref, tile, sems):
    num_clips, d = col_ref.shape
    hw = x_ref.shape[0]
    w = hw // num_clips
    b = out_ref.shape[0]

    k_iota = jax.lax.broadcasted_iota(jnp.int32, (hw, num_clips), 1)
    p_iota = jax.lax.broadcasted_iota(jnp.int32, (hw, num_clips), 0)

    onehot_col = (x_ref[:] == k_iota).astype(jnp.float32)
    onehot_row = ((p_iota // w) == k_iota).astype(jnp.float32)

    dn = (((1,), (0,)), ((), ()))

    n_blocks = 4
    rows = hw // n_blocks
    copies = []
    for blk in range(n_blocks):
        rs = pl.ds(blk * rows, rows)
        tile[rs, :d] = jax.lax.dot_general(
            onehot_col[blk * rows:(blk + 1) * rows, :], col_ref[:], dn,
            preferred_element_type=jnp.float32,
            precision=jax.lax.Precision.HIGHEST)
        tile[rs, d:] = jax.lax.dot_general(
            onehot_row[blk * rows:(blk + 1) * rows, :], row_ref[:], dn,
            preferred_element_type=jnp.float32,
            precision=jax.lax.Precision.HIGHEST)
        for i in range(b):
            c = pltpu.make_async_copy(tile.at[rs], out_ref.at[i, rs],
                                      sems.at[i])
            c.start()
            copies.append(c)
    for c in copies:
        c.wait()


def kernel(x, col_embed, row_embed):
    h, w = x.shape
    num_clips, d = col_embed.shape
    b = h
    hw = h * w

    x_col = x.reshape(hw, 1)

    out_pm = pl.pallas_call(
        _pos_embed_kernel,
        in_specs=[
            pl.BlockSpec(memory_space=pltpu.MemorySpace.VMEM),
            pl.BlockSpec(memory_space=pltpu.MemorySpace.VMEM),
            pl.BlockSpec(memory_space=pltpu.MemorySpace.VMEM),
        ],
        out_specs=pl.BlockSpec(memory_space=pltpu.MemorySpace.HBM),
        out_shape=jax.ShapeDtypeStruct((b, hw, 2 * d), jnp.float32),
        scratch_shapes=[
            pltpu.VMEM((hw, 2 * d), jnp.float32),
            pltpu.SemaphoreType.DMA((b,)),
        ],
    )(x_col, col_embed, row_embed)

    return out_pm.reshape(b, h, w, 2 * d).transpose(0, 3, 1, 2)


# speedup vs baseline: 1.0196x; 1.0196x over previous
import jax
import jax.numpy as jnp
from jax.experimental import pallas as pl
from jax.experimental.pallas import tpu as pltpu


def _pos_embed_kernel(x_ref, col_ref, row_ref, out_ref, tile, sems):
    num_clips, d = col_ref.shape
    hw = x_ref.shape[0]
    w = hw // num_clips
    b = out_ref.shape[0]

    k_iota = jax.lax.broadcasted_iota(jnp.int32, (hw, num_clips), 1)
    p_iota = jax.lax.broadcasted_iota(jnp.int32, (hw, num_clips), 0)

    onehot_col = (x_ref[:] == k_iota).astype(jnp.float32)
    onehot_row = ((p_iota // w) == k_iota).astype(jnp.float32)

    dn = (((1,), (0,)), ((), ()))

    n_blocks = 16
    rows = hw // n_blocks
    copies = []
    for blk in range(n_blocks):
        rs = pl.ds(blk * rows, rows)
        tile[rs, :d] = jax.lax.dot_general(
            onehot_col[blk * rows:(blk + 1) * rows, :], col_ref[:], dn,
            preferred_element_type=jnp.float32,
            precision=jax.lax.Precision.HIGHEST)
        tile[rs, d:] = jax.lax.dot_general(
            onehot_row[blk * rows:(blk + 1) * rows, :], row_ref[:], dn,
            preferred_element_type=jnp.float32,
            precision=jax.lax.Precision.HIGHEST)
        for i in range(b):
            c = pltpu.make_async_copy(tile.at[rs], out_ref.at[i, rs],
                                      sems.at[i])
            c.start()
            copies.append(c)
    for c in copies:
        c.wait()


def kernel(x, col_embed, row_embed):
    h, w = x.shape
    num_clips, d = col_embed.shape
    b = h
    hw = h * w

    x_col = x.reshape(hw, 1)

    out_pm = pl.pallas_call(
        _pos_embed_kernel,
        in_specs=[
            pl.BlockSpec(memory_space=pltpu.MemorySpace.VMEM),
            pl.BlockSpec(memory_space=pltpu.MemorySpace.VMEM),
            pl.BlockSpec(memory_space=pltpu.MemorySpace.VMEM),
        ],
        out_specs=pl.BlockSpec(memory_space=pltpu.MemorySpace.HBM),
        out_shape=jax.ShapeDtypeStruct((b, hw, 2 * d), jnp.float32),
        scratch_shapes=[
            pltpu.VMEM((hw, 2 * d), jnp.float32),
            pltpu.SemaphoreType.DMA((b,)),
        ],
    )(x_col, col_embed, row_embed)

    return out_pm.reshape(b, h, w, 2 * d).transpose(0, 3, 1, 2)


# speedup vs baseline: 1.0347x; 1.0148x over previous
import jax
import jax.numpy as jnp
from jax.experimental import pallas as pl
from jax.experimental.pallas import tpu as pltpu


def _pos_embed_kernel(x_ref, col_ref, row_ref, out_ref, tile, sems):
    num_clips, d = col_ref.shape
    hw = x_ref.shape[0]
    w = hw // num_clips
    b = out_ref.shape[0]

    k_iota = jax.lax.broadcasted_iota(jnp.int32, (hw, num_clips), 1)
    p_iota = jax.lax.broadcasted_iota(jnp.int32, (hw, num_clips), 0)

    onehot_col = (x_ref[:] == k_iota).astype(jnp.float32)
    onehot_row = ((p_iota // w) == k_iota).astype(jnp.float32)

    dn = (((1,), (0,)), ((), ()))

    n_blocks = 32
    rows = hw // n_blocks
    copies = []
    for blk in range(n_blocks):
        rs = pl.ds(blk * rows, rows)
        tile[rs, :d] = jax.lax.dot_general(
            onehot_col[blk * rows:(blk + 1) * rows, :], col_ref[:], dn,
            preferred_element_type=jnp.float32,
            precision=jax.lax.Precision.HIGHEST)
        tile[rs, d:] = jax.lax.dot_general(
            onehot_row[blk * rows:(blk + 1) * rows, :], row_ref[:], dn,
            preferred_element_type=jnp.float32,
            precision=jax.lax.Precision.HIGHEST)
        for i in range(b):
            c = pltpu.make_async_copy(tile.at[rs], out_ref.at[i, rs],
                                      sems.at[i])
            c.start()
            copies.append(c)
    for c in copies:
        c.wait()


def kernel(x, col_embed, row_embed):
    h, w = x.shape
    num_clips, d = col_embed.shape
    b = h
    hw = h * w

    x_col = x.reshape(hw, 1)

    out_pm = pl.pallas_call(
        _pos_embed_kernel,
        in_specs=[
            pl.BlockSpec(memory_space=pltpu.MemorySpace.VMEM),
            pl.BlockSpec(memory_space=pltpu.MemorySpace.VMEM),
            pl.BlockSpec(memory_space=pltpu.MemorySpace.VMEM),
        ],
        out_specs=pl.BlockSpec(memory_space=pltpu.MemorySpace.HBM),
        out_shape=jax.ShapeDtypeStruct((b, hw, 2 * d), jnp.float32),
        scratch_shapes=[
            pltpu.VMEM((hw, 2 * d), jnp.float32),
            pltpu.SemaphoreType.DMA((b,)),
        ],
    )(x_col, col_embed, row_embed)

    return out_pm.reshape(b, h, w, 2 * d).transpose(0, 3, 1, 2)
